# baseline (speedup 1.0000x reference)
import functools

import jax
import jax.numpy as jnp
from jax import lax
from jax.experimental import pallas as pl
from jax.experimental.pallas import tpu as pltpu
from jax.experimental.pallas import tpu_sc as plsc

_B = 16384
_C = 1000
_CP = 1024
_NC = 2
_NS = 16
_NW = _NC * _NS
_RPW = _B // _NW
_L = 16
_G = 4
_MARGIN = 1.0
_INF = float("inf")


def _sc_body(x_hbm, c_hbm, t_hbm, out_hbm, x_v, t_v, c_v, o_v):
    wid = lax.axis_index("s") * _NC + lax.axis_index("c")
    base = wid * _RPW
    pltpu.sync_copy(x_hbm.at[pl.ds(base, _RPW)], x_v)
    pltpu.sync_copy(t_hbm.at[pl.ds(base, _RPW)], t_v)
    pltpu.sync_copy(c_hbm, c_v)

    acc = jnp.zeros((_L,), jnp.float32)
    for g in range(_RPW // (_L * _G)):
        xs = [x_v[pl.ds((g * _G + k) * _L, _L)] for k in range(_G)]
        ts = [t_v[pl.ds((g * _G + k) * _L, _L)] for k in range(_G)]

        def jbody(jj, m1s, xs=xs):
            m1s = list(m1s)
            cblk = c_v[pl.ds(jj * _L, _L)]
            for u in range(_L):
                cj = cblk[u]
                for k in range(_G):
                    m1s[k] = jnp.minimum(m1s[k], jnp.abs(xs[k] - cj))
            return tuple(m1s)

        init = tuple(jnp.full((_L,), _INF, jnp.float32) for _ in range(_G))
        m1s = lax.fori_loop(0, _CP // _L, jbody, init)
        for k in range(_G):
            own = plsc.load_gather(c_v, [ts[k]])
            d_own = jnp.abs(xs[k] - own)
            hit = d_own == m1s[k]
            nhit = plsc.all_reduce_population_count(hit)

            def fix(xk=xs[k], tk=ts[k]):
                def fbody(jj, m):
                    cblk = c_v[pl.ds(jj * _L, _L)]
                    jbase = jj * _L
                    for u in range(_L):
                        d = jnp.abs(xk - cblk[u])
                        m = jnp.minimum(m, jnp.where(tk == jbase + u, _INF, d))
                    return m

                return lax.fori_loop(
                    0, _CP // _L, fbody, jnp.full((_L,), _INF, jnp.float32)
                )

            push = lax.cond(nhit[0] > 0, fix, lambda mk=m1s[k]: mk)
            pull = d_own + _MARGIN
            acc = acc + jnp.maximum(pull - push, 0.0)

    o_v[...] = acc
    pltpu.sync_copy(o_v, out_hbm.at[pl.ds(wid * _L, _L)])


_sc_call = functools.partial(
    pl.kernel,
    out_type=jax.ShapeDtypeStruct((_NW * _L,), jnp.float32),
    mesh=plsc.VectorSubcoreMesh(core_axis_name="c", subcore_axis_name="s"),
    compiler_params=pltpu.CompilerParams(needs_layout_passes=False),
    scratch_types=[
        pltpu.VMEM((_RPW,), jnp.float32),
        pltpu.VMEM((_RPW,), jnp.int32),
        pltpu.VMEM((_CP,), jnp.float32),
        pltpu.VMEM((_L,), jnp.float32),
    ],
)(_sc_body)


def kernel(x, centers, transform_inds):
    xf = x.reshape(_B)
    cf = jnp.concatenate(
        [centers.reshape(_C), jnp.full((_CP - _C,), _INF, jnp.float32)]
    )
    partial = _sc_call(xf, cf, transform_inds)
    return jnp.sum(partial) / _B


# speedup vs baseline: 1.1551x; 1.1551x over previous
import functools

import jax
import jax.numpy as jnp
from jax import lax
from jax.experimental import pallas as pl
from jax.experimental.pallas import tpu as pltpu
from jax.experimental.pallas import tpu_sc as plsc

_B = 16384
_C = 1000
_CP = 1024
_NC = 2
_NS = 16
_NW = _NC * _NS
_RPW = _B // _NW
_L = 16
_G = 8
_MARGIN = 1.0
_INF = float("inf")


def _sc_body(x_hbm, c_hbm, t_hbm, out_hbm, x_v, t_v, c_v, o_v):
    wid = lax.axis_index("s") * _NC + lax.axis_index("c")
    base = wid * _RPW
    pltpu.sync_copy(x_hbm.at[pl.ds(base, _RPW)], x_v)
    pltpu.sync_copy(t_hbm.at[pl.ds(base, _RPW)], t_v)
    pltpu.sync_copy(c_hbm, c_v)

    acc = jnp.zeros((_L,), jnp.float32)
    for g in range(_RPW // (_L * _G)):
        xs = [x_v[pl.ds((g * _G + k) * _L, _L)] for k in range(_G)]
        ts = [t_v[pl.ds((g * _G + k) * _L, _L)] for k in range(_G)]

        def jbody(jj, carry, xs=xs):
            m1s, m2s = list(carry[0]), list(carry[1])
            cblk = c_v[pl.ds(jj * _L, _L)]
            for u in range(_L):
                cj = cblk[u]
                for k in range(_G):
                    d = jnp.abs(xs[k] - cj)
                    m2s[k] = jnp.minimum(m2s[k], jnp.maximum(m1s[k], d))
                    m1s[k] = jnp.minimum(m1s[k], d)
            return tuple(m1s), tuple(m2s)

        init = (
            tuple(jnp.full((_L,), _INF, jnp.float32) for _ in range(_G)),
            tuple(jnp.full((_L,), _INF, jnp.float32) for _ in range(_G)),
        )
        m1s, m2s = lax.fori_loop(0, _CP // _L, jbody, init)
        for k in range(_G):
            own = plsc.load_gather(c_v, [ts[k]])
            d_own = jnp.abs(xs[k] - own)
            push = jnp.where(d_own == m1s[k], m2s[k], m1s[k])
            pull = d_own + _MARGIN
            acc = acc + jnp.maximum(pull - push, 0.0)

    o_v[...] = acc
    pltpu.sync_copy(o_v, out_hbm.at[pl.ds(wid * _L, _L)])


_sc_call = functools.partial(
    pl.kernel,
    out_type=jax.ShapeDtypeStruct((_NW * _L,), jnp.float32),
    mesh=plsc.VectorSubcoreMesh(core_axis_name="c", subcore_axis_name="s"),
    compiler_params=pltpu.CompilerParams(needs_layout_passes=False),
    scratch_types=[
        pltpu.VMEM((_RPW,), jnp.float32),
        pltpu.VMEM((_RPW,), jnp.int32),
        pltpu.VMEM((_CP,), jnp.float32),
        pltpu.VMEM((_L,), jnp.float32),
    ],
)(_sc_body)


def kernel(x, centers, transform_inds):
    xf = x.reshape(_B)
    cf = jnp.concatenate(
        [centers.reshape(_C), jnp.full((_CP - _C,), _INF, jnp.float32)]
    )
    partial = _sc_call(xf, cf, transform_inds)
    return jnp.sum(partial) / _B


# speedup vs baseline: 1.6805x; 1.4549x over previous
import functools

import jax
import jax.numpy as jnp
from jax import lax
from jax.experimental import pallas as pl
from jax.experimental.pallas import tpu as pltpu
from jax.experimental.pallas import tpu_sc as plsc

_B = 16384
_C = 1000
_CP = 1024
_NC = 2
_NS = 16
_NW = _NC * _NS
_RPW = _B // _NW
_L = 16
_NB = _CP // _L
_G = 4
_MARGIN = 1.0
_INF = float("inf")


def _sc_body(x_hbm, c_hbm, t_hbm, out_hbm, x_v, t_v, c_v, ck_v, ci_v, o_v):
    wid = lax.axis_index("s") * _NC + lax.axis_index("c")
    base = wid * _RPW
    pltpu.sync_copy(x_hbm.at[pl.ds(base, _RPW)], x_v)
    pltpu.sync_copy(t_hbm.at[pl.ds(base, _RPW)], t_v)
    pltpu.sync_copy(c_hbm, c_v)
    pltpu.sync_copy(c_hbm, ck_v)

    lane = lax.iota(jnp.int32, _L)

    def ibody(b, _):
        ci_v[pl.ds(b * _L, _L)] = lane + b * _L
        return 0

    lax.fori_loop(0, _NB, ibody, 0)

    def seed(q, _):
        o0 = q * (2 * _L)
        k0, v0 = ck_v[pl.ds(o0, _L)], ci_v[pl.ds(o0, _L)]
        ks0, vs0 = plsc.sort_key_val(k0, v0)
        ck_v[pl.ds(o0, _L)] = ks0
        ci_v[pl.ds(o0, _L)] = vs0
        o1 = o0 + _L
        k1, v1 = ck_v[pl.ds(o1, _L)], ci_v[pl.ds(o1, _L)]
        ks1, vs1 = plsc.sort_key_val(k1, v1, descending=True)
        ck_v[pl.ds(o1, _L)] = ks1
        ci_v[pl.ds(o1, _L)] = vs1
        return 0

    lax.fori_loop(0, _NB // 2, seed, 0)

    def _ce(r, p, ascv):
        ka, kb = ck_v[pl.ds(r * _L, _L)], ck_v[pl.ds(p * _L, _L)]
        va, vb = ci_v[pl.ds(r * _L, _L)], ci_v[pl.ds(p * _L, _L)]
        cond = jnp.logical_xor(ka > kb, ascv != 0)
        ck_v[pl.ds(r * _L, _L)] = jnp.where(cond, ka, kb)
        ck_v[pl.ds(p * _L, _L)] = jnp.where(cond, kb, ka)
        ci_v[pl.ds(r * _L, _L)] = jnp.where(cond, va, vb)
        ci_v[pl.ds(p * _L, _L)] = jnp.where(cond, vb, va)

    _U = 4
    for kr in (2, 4, 8, 16, 32, 64):
        sr = kr // 2
        while sr >= 1:
            sh = sr.bit_length() - 1

            def stage(i, _, sr=sr, sh=sh, kr=kr):
                for u in range(_U):
                    q = i * _U + u
                    r = ((q >> sh) << (sh + 1)) | (q & (sr - 1))
                    asc = jnp.broadcast_to(
                        jnp.bitwise_and(r, kr), (_L,)
                    ) == 0
                    _ce(r, r + sr, jnp.where(asc, 1, 0))
                return 0

            lax.fori_loop(0, (_NB // 2) // _U, stage, 0)
            sr //= 2

        if kr < _NB:

            def cleanup(i, _, kr=kr):
                for u in range(_U):
                    q = i * _U + u
                    off = q * _L
                    k, v = ck_v[pl.ds(off, _L)], ci_v[pl.ds(off, _L)]
                    ks, vs = plsc.sort_key_val(k, v)
                    asc = jnp.broadcast_to(
                        jnp.bitwise_and(q, kr), (_L,)
                    ) == 0
                    ck_v[pl.ds(off, _L)] = jnp.where(
                        asc, ks, lax.rev(ks, (0,))
                    )
                    ci_v[pl.ds(off, _L)] = jnp.where(
                        asc, vs, lax.rev(vs, (0,))
                    )
                return 0

            lax.fori_loop(0, _NB // _U, cleanup, 0)
        else:

            def cleanup_last(i, _):
                for u in range(_U):
                    off = (i * _U + u) * _L
                    k, v = ck_v[pl.ds(off, _L)], ci_v[pl.ds(off, _L)]
                    ks, vs = plsc.sort_key_val(k, v)
                    ck_v[pl.ds(off, _L)] = ks
                    ci_v[pl.ds(off, _L)] = vs
                return 0

            lax.fori_loop(0, _NB // _U, cleanup_last, 0)

    acc = jnp.zeros((_L,), jnp.float32)
    for g in range(_RPW // (_L * _G)):
        xs = [x_v[pl.ds((g * _G + k) * _L, _L)] for k in range(_G)]
        ts = [t_v[pl.ds((g * _G + k) * _L, _L)] for k in range(_G)]
        poss = [jnp.zeros((_L,), jnp.int32) for _ in range(_G)]
        s = _CP // 2
        while s >= 1:
            for k in range(_G):
                probe = poss[k] + (s - 1)
                key = plsc.load_gather(ck_v, [probe])
                poss[k] = poss[k] + jnp.where(key < xs[k], s, 0)
            s //= 2
        for k in range(_G):
            push = jnp.full((_L,), _INF, jnp.float32)
            for dq in (-2, -1, 0, 1):
                q = poss[k] + dq
                qc = jnp.maximum(q, 0)
                key = plsc.load_gather(ck_v, [qc])
                idx = plsc.load_gather(ci_v, [qc])
                d = jnp.abs(xs[k] - key)
                ok = jnp.logical_and(q >= 0, idx != ts[k])
                push = jnp.minimum(push, jnp.where(ok, d, _INF))
            own = plsc.load_gather(c_v, [ts[k]])
            d_own = jnp.abs(xs[k] - own)
            acc = acc + jnp.maximum(d_own + _MARGIN - push, 0.0)

    o_v[...] = acc
    pltpu.sync_copy(o_v, out_hbm.at[pl.ds(wid * _L, _L)])


_sc_call = functools.partial(
    pl.kernel,
    out_type=jax.ShapeDtypeStruct((_NW * _L,), jnp.float32),
    mesh=plsc.VectorSubcoreMesh(core_axis_name="c", subcore_axis_name="s"),
    compiler_params=pltpu.CompilerParams(needs_layout_passes=False),
    scratch_types=[
        pltpu.VMEM((_RPW,), jnp.float32),
        pltpu.VMEM((_RPW,), jnp.int32),
        pltpu.VMEM((_CP,), jnp.float32),
        pltpu.VMEM((_CP,), jnp.float32),
        pltpu.VMEM((_CP,), jnp.int32),
        pltpu.VMEM((_L,), jnp.float32),
    ],
)(_sc_body)


def kernel(x, centers, transform_inds):
    xf = x.reshape(_B)
    cf = jnp.concatenate(
        [centers.reshape(_C), jnp.full((_CP - _C,), _INF, jnp.float32)]
    )
    partial = _sc_call(xf, cf, transform_inds)
    return jnp.sum(partial) / _B
